# baseline (speedup 1.0000x reference)
import functools

import numpy as np

import jax
import jax.numpy as jnp
from jax import lax
from jax.experimental import pallas as pl
from jax.experimental.pallas import tpu as pltpu
from jax.experimental.pallas import tpu_sc as plsc

V = 100000
D = 128
H = 256
NP = 1000
NPP = 1024
B = 4096
GROUP = 200
GCH = 112

_W1_MAP = np.concatenate(
    [np.concatenate([32 * k + np.arange(0, 32, 2),
                     32 * k + np.arange(1, 32, 2)]) for k in range(4)])


def _make_pool_kernel():
    info = plsc.get_sparse_core_info()
    nc, ns = info.num_cores, info.num_subcores
    nw = nc * ns
    assert B % nw == 0
    bpw = B // nw

    mesh = plsc.VectorSubcoreMesh(core_axis_name="c", subcore_axis_name="s")

    @functools.partial(
        pl.kernel,
        mesh=mesh,
        out_type=jax.ShapeDtypeStruct((B, D), jnp.float32),
        scratch_types=[
            pltpu.VMEM((bpw, 2, GCH), jnp.int32),
            pltpu.VMEM((2, 2, GCH, D), jnp.float32),
            pltpu.VMEM((bpw, D), jnp.float32),
            pltpu.SemaphoreType.DMA,
            pltpu.SemaphoreType.DMA,
        ],
        compiler_params=pltpu.CompilerParams(use_tc_tiling_on_sc=False),
    )
    def pool(idx_hbm, table_hbm, out_hbm, idx_v, rows_v, out_v, sem0, sem1):
        wid = lax.axis_index("s") * nc + lax.axis_index("c")
        base = wid * bpw
        sems = (sem0, sem1)

        pltpu.sync_copy(idx_hbm.at[pl.ds(base, bpw)], idx_v)

        def fire(row, slot):
            pltpu.async_copy(table_hbm.at[idx_v.at[row, 0]],
                             rows_v.at[slot, 0], sems[slot])
            pltpu.async_copy(table_hbm.at[idx_v.at[row, 1]],
                             rows_v.at[slot, 1], sems[slot])

        def drain(row, slot):
            pltpu.make_async_copy(table_hbm.at[idx_v.at[row, 0]],
                                  rows_v.at[slot, 0], sems[slot]).wait()
            pltpu.make_async_copy(table_hbm.at[idx_v.at[row, 1]],
                                  rows_v.at[slot, 1], sems[slot]).wait()

        fire(0, 0)
        fire(1, 1)

        def body(i, _):
            e = i * 2
            for slot in range(2):
                row = e + slot
                drain(row, slot)

                def make_rbody(g):
                    def rbody(r2, accs):
                        r = r2 * 2
                        new = list(accs)
                        for dr in range(2):
                            for k in range(8):
                                w = rows_v[slot, g, r + dr,
                                           pl.ds(k * 16, 16)]
                                new[k] = new[k] + w
                        return tuple(new)
                    return rbody

                accs = tuple(jnp.zeros((16,), jnp.float32) for _ in range(8))
                accs = lax.fori_loop(0, GCH // 2, make_rbody(0), accs)
                accs = lax.fori_loop(0, (GROUP - GCH) // 2, make_rbody(1),
                                     accs)
                for c in range(8):
                    out_v[row, pl.ds(c * 16, 16)] = accs[c]

                @pl.when(row + 2 < bpw)
                def _():
                    fire(row + 2, slot)
            return 0

        lax.fori_loop(0, bpw // 2, body, 0)
        pltpu.sync_copy(out_v, out_hbm.at[pl.ds(base, bpw)])

    return pool


def _mlp_body(s_ref, w1_ref, b1_ref, w2_ref, b2_ref, out_ref):
    s = s_ref[...]
    h = jax.nn.sigmoid(
        jnp.dot(s, w1_ref[...], preferred_element_type=jnp.float32)
        + b1_ref[...])
    logits = (jnp.dot(h, w2_ref[...], preferred_element_type=jnp.float32)
              + b2_ref[...])
    m = jnp.max(logits, axis=-1, keepdims=True)
    lse = jnp.log(jnp.sum(jnp.exp(logits - m), axis=-1, keepdims=True)) + m
    out_ref[...] = logits - lse


def _mlp(pooled, w1, b1, w2, b2):
    bm = 512
    grid = (B // bm,)
    return pl.pallas_call(
        _mlp_body,
        grid=grid,
        in_specs=[
            pl.BlockSpec((bm, D), lambda i: (i, 0)),
            pl.BlockSpec((D, H), lambda i: (0, 0)),
            pl.BlockSpec((1, H), lambda i: (0, 0)),
            pl.BlockSpec((H, NPP), lambda i: (0, 0)),
            pl.BlockSpec((1, NPP), lambda i: (0, 0)),
        ],
        out_specs=pl.BlockSpec((bm, NPP), lambda i: (i, 0)),
        out_shape=jax.ShapeDtypeStruct((B, NPP), jnp.float32),
    )(pooled, w1, b1, w2, b2)


def kernel(x, table, W1, b1, W2, b2):
    idx = jnp.pad(x.reshape(B, GROUP),
                  ((0, 0), (0, 2 * GCH - GROUP))).reshape(B, 2, GCH)
    pooled = _make_pool_kernel()(idx, table)
    w1p = W1
    w2p = jnp.pad(W2, ((0, 0), (0, NPP - NP)))
    b2p = jnp.pad(b2, (0, NPP - NP), constant_values=-1e30)
    out = _mlp(pooled, w1p, b1.reshape(1, H), w2p, b2p.reshape(1, NPP))
    return out[:, :NP]


# speedup vs baseline: 20.4354x; 20.4354x over previous
import functools

import jax
import jax.numpy as jnp
from jax import lax
from jax.experimental import pallas as pl
from jax.experimental.pallas import tpu as pltpu
from jax.experimental.pallas import tpu_sc as plsc

V = 100000
D = 128
H = 256
NP = 1000
NPP = 1024
B = 4096
GROUP = 200
HALF = GROUP // 2
NSLOT = 3
RUNROLL = 8


def _make_pool_kernel():
    info = plsc.get_sparse_core_info()
    nc, ns = info.num_cores, info.num_subcores
    nw = nc * ns
    assert B % nw == 0
    bpw = B // nw

    mesh = plsc.VectorSubcoreMesh(core_axis_name="c", subcore_axis_name="s")

    @functools.partial(
        pl.kernel,
        mesh=mesh,
        out_type=jax.ShapeDtypeStruct((B, D), jnp.float32),
        scratch_types=[
            pltpu.VMEM((bpw, 2, HALF), jnp.int32),
            pltpu.VMEM((NSLOT, GROUP, D), jnp.float32),
            pltpu.VMEM((bpw, D), jnp.float32),
            pltpu.SemaphoreType.DMA,
            pltpu.SemaphoreType.DMA,
            pltpu.SemaphoreType.DMA,
        ],
    )
    def pool(idx_hbm, table_hbm, out_hbm, idx_v, rows_v, out_v, *sems):
        wid = lax.axis_index("s") * nc + lax.axis_index("c")
        base = wid * bpw

        pltpu.sync_copy(idx_hbm.at[pl.ds(base, bpw)], idx_v)

        def fire(row, slot):
            pltpu.async_copy(table_hbm.at[idx_v.at[row, 0]],
                             rows_v.at[slot, pl.ds(0, HALF)], sems[slot])
            pltpu.async_copy(table_hbm.at[idx_v.at[row, 1]],
                             rows_v.at[slot, pl.ds(HALF, HALF)], sems[slot])

        def drain(row, slot):
            pltpu.make_async_copy(table_hbm.at[idx_v.at[row, 0]],
                                  rows_v.at[slot, pl.ds(0, HALF)],
                                  sems[slot]).wait()
            pltpu.make_async_copy(table_hbm.at[idx_v.at[row, 1]],
                                  rows_v.at[slot, pl.ds(HALF, HALF)],
                                  sems[slot]).wait()

        for s in range(NSLOT):
            fire(s, s)

        def body(i, _):
            row = i * NSLOT
            for slot in range(NSLOT):
                drain(row + slot, slot)

                def rbody(r8, accs):
                    r = r8 * RUNROLL
                    new = list(accs)
                    for dr in range(RUNROLL):
                        for k in range(8):
                            new[k] = new[k] + rows_v[slot, r + dr,
                                                     pl.ds(k * 16, 16)]
                    return tuple(new)

                accs = lax.fori_loop(
                    0, GROUP // RUNROLL, rbody,
                    tuple(jnp.zeros((16,), jnp.float32) for _ in range(8)),
                    unroll=1)
                for k in range(8):
                    out_v[row + slot, pl.ds(k * 16, 16)] = accs[k]

                @pl.when(row + slot + NSLOT < bpw)
                def _():
                    fire(row + slot + NSLOT, slot)
            return 0

        nfull = bpw // NSLOT
        lax.fori_loop(0, nfull, body, 0)
        for t in range(nfull * NSLOT, bpw):
            slot = t % NSLOT
            drain(t, slot)

            def rtail(r8, accs):
                r = r8 * RUNROLL
                new = list(accs)
                for dr in range(RUNROLL):
                    for k in range(8):
                        new[k] = new[k] + rows_v[slot, r + dr,
                                                 pl.ds(k * 16, 16)]
                return tuple(new)

            accs = lax.fori_loop(
                0, GROUP // RUNROLL, rtail,
                tuple(jnp.zeros((16,), jnp.float32) for _ in range(8)),
                unroll=1)
            for k in range(8):
                out_v[t, pl.ds(k * 16, 16)] = accs[k]

        pltpu.sync_copy(out_v, out_hbm.at[pl.ds(base, bpw)])

    return pool


def _mlp_body(s_ref, w1_ref, b1_ref, w2_ref, b2_ref, out_ref):
    s = s_ref[...]
    h = jax.nn.sigmoid(
        jnp.dot(s, w1_ref[...], preferred_element_type=jnp.float32)
        + b1_ref[...])
    logits = (jnp.dot(h, w2_ref[...], preferred_element_type=jnp.float32)
              + b2_ref[...])
    m = jnp.max(logits, axis=-1, keepdims=True)
    lse = jnp.log(jnp.sum(jnp.exp(logits - m), axis=-1, keepdims=True)) + m
    out_ref[...] = logits - lse


def _mlp(pooled, w1, b1, w2, b2):
    bm = 512
    grid = (B // bm,)
    return pl.pallas_call(
        _mlp_body,
        grid=grid,
        in_specs=[
            pl.BlockSpec((bm, D), lambda i: (i, 0)),
            pl.BlockSpec((D, H), lambda i: (0, 0)),
            pl.BlockSpec((1, H), lambda i: (0, 0)),
            pl.BlockSpec((H, NPP), lambda i: (0, 0)),
            pl.BlockSpec((1, NPP), lambda i: (0, 0)),
        ],
        out_specs=pl.BlockSpec((bm, NPP), lambda i: (i, 0)),
        out_shape=jax.ShapeDtypeStruct((B, NPP), jnp.float32),
    )(pooled, w1, b1, w2, b2)


def kernel(x, table, W1, b1, W2, b2):
    idx = x.reshape(B, 2, HALF)
    pooled = _make_pool_kernel()(idx, table)
    w2p = jnp.pad(W2, ((0, 0), (0, NPP - NP)))
    b2p = jnp.pad(b2, (0, NPP - NP), constant_values=-1e30)
    out = _mlp(pooled, W1, b1.reshape(1, H), w2p, b2p.reshape(1, NPP))
    return out[:, :NP]
